# baseline (speedup 1.0000x reference)
import functools

import jax
import jax.numpy as jnp
from jax import lax
from jax.experimental import pallas as pl
from jax.experimental.pallas import tpu as pltpu
from jax.experimental.pallas import tpu_sc as plsc

N = 10000
E = 320000
F = 128
H = 16
C = 4

NC = 2
NS = 16
NW = NC * NS

CHUNK = 200
NCH = 50
EPW = NCH * CHUNK

NP = 10240
RPS = NP // NS

NBUF = 5
AHEAD = 4

_mesh = plsc.VectorSubcoreMesh(
    core_axis_name="c", subcore_axis_name="s", num_cores=NC, num_subcores=NS
)

_sc_params = pltpu.CompilerParams(use_tc_tiling_on_sc=False)


def _stripe_out(zb_v, acc_sh, out_hbm, cid, sid):
    pltpu.sync_copy(acc_sh.at[pl.ds(sid * RPS, RPS)], zb_v)
    pltpu.sync_copy(zb_v, out_hbm.at[cid, sid])


def _zero_acc(zb_v, acc_sh, sid):
    zero_row = jnp.zeros((H,), jnp.float32)

    def zfill(i, c):
        zb_v[i, :] = zero_row
        return c
    lax.fori_loop(0, RPS, zfill, 0)
    pltpu.sync_copy(zb_v, acc_sh.at[pl.ds(sid * RPS, RPS)])


@functools.partial(
    pl.kernel,
    out_type=jax.ShapeDtypeStruct((NC, NS, RPS, H), jnp.float32),
    mesh=_mesh,
    scratch_types=[
        pltpu.VMEM((NCH, CHUNK), jnp.int32),
        pltpu.VMEM((CHUNK, H), jnp.float32),
        pltpu.VMEM((RPS, H), jnp.float32),
        pltpu.VMEM_SHARED((NP, H), jnp.float32),
        pltpu.SemaphoreType.DMA,
    ],
    compiler_params=_sc_params,
)
def _deg_kernel(ei_hbm, out_hbm, coli_v, ones_v, zb_v, acc_sh, sem_s):
    cid = lax.axis_index("c")
    sid = lax.axis_index("s")
    wid = cid * NS + sid

    one_row = jnp.full((H,), 1.0, jnp.float32)

    def fill(i, c):
        ones_v[i, :] = one_row
        return c
    lax.fori_loop(0, CHUNK, fill, 0)

    _zero_acc(zb_v, acc_sh, sid)
    pltpu.sync_copy(ei_hbm.at[1, wid], coli_v)
    plsc.subcore_barrier()

    def scat(j):
        return pltpu.make_async_copy(ones_v, acc_sh.at[coli_v.at[j]], sem_s)

    LAG = 4

    def sc(j, c):
        scat(j).start(add=True)

        @pl.when(j >= LAG)
        def _():
            scat(j - LAG).wait()
        return c
    lax.fori_loop(0, NCH, sc, 0)

    def drain(j, c):
        scat(NCH - LAG + j).wait()
        return c
    lax.fori_loop(0, LAG, drain, 0)

    plsc.subcore_barrier()
    _stripe_out(zb_v, acc_sh, out_hbm, cid, sid)


@functools.partial(
    pl.kernel,
    out_type=jax.ShapeDtypeStruct((NC, NS, RPS, H), jnp.float32),
    mesh=_mesh,
    scratch_types=[
        pltpu.VMEM((NCH, CHUNK), jnp.int32),
        pltpu.VMEM((NCH, CHUNK), jnp.int32),
        [pltpu.VMEM((CHUNK, H), jnp.float32)] * NBUF,
        pltpu.VMEM((RPS, H), jnp.float32),
        pltpu.VMEM_SHARED((NP, H), jnp.float32),
        pltpu.SemaphoreType.DMA,
        pltpu.SemaphoreType.DMA,
    ],
    compiler_params=_sc_params,
)
def _prop_kernel(g_hbm, ei_hbm, out_hbm,
                 rowi_v, coli_v, bufs, zb_v, acc_sh, sem_g, sem_s):
    cid = lax.axis_index("c")
    sid = lax.axis_index("s")
    wid = cid * NS + sid

    _zero_acc(zb_v, acc_sh, sid)
    pltpu.sync_copy(ei_hbm.at[0, wid], rowi_v)
    pltpu.sync_copy(ei_hbm.at[1, wid], coli_v)
    plsc.subcore_barrier()

    def gat(j, b):
        return pltpu.make_async_copy(g_hbm.at[rowi_v.at[j]], bufs[b], sem_g)

    def scat(j, b):
        return pltpu.make_async_copy(bufs[b], acc_sh.at[coli_v.at[j]], sem_s)

    for j in range(AHEAD):
        gat(j, j).start()

    def step(g, c):
        for b in range(NBUF):
            j = g * NBUF + b
            bp = (b + AHEAD) % NBUF
            back = NBUF - AHEAD
            if b >= back:
                scat(j - back, bp).wait()
            else:
                @pl.when(j >= back)
                def _():
                    scat(j - back, bp).wait()

            @pl.when(j + AHEAD < NCH)
            def _():
                gat(j + AHEAD, bp).start()

            gat(j, b).wait()
            scat(j, b).start(add=True)
        return c
    lax.fori_loop(0, NCH // NBUF, step, 0)

    for k in range(NBUF - AHEAD):
        j = NCH - (NBUF - AHEAD) + k
        scat(j, j % NBUF).wait()

    plsc.subcore_barrier()
    _stripe_out(zb_v, acc_sh, out_hbm, cid, sid)


PK = NP // 8
BNP = PK // 2


def _mm(a, b):
    return lax.dot_general(a, b, (((1,), (0,)), ((), ())),
                           preferred_element_type=jnp.float32)


def _dinv(dacc_ref):
    return lax.rsqrt(dacc_ref[0] + dacc_ref[1] + 1.0)


def _tc1_body(x2_ref, w1_ref, dacc_ref, g_ref):
    g_ref[...] = _mm(x2_ref[...], w1_ref[...]) * _dinv(dacc_ref)


_tc1 = pl.pallas_call(
    _tc1_body,
    grid=(PK // BNP,),
    in_specs=[
        pl.BlockSpec((BNP, 8 * F), lambda i: (i, 0)),
        pl.BlockSpec((8 * F, 128), lambda i: (0, 0)),
        pl.BlockSpec((NC, BNP, 128), lambda i: (0, i, 0)),
    ],
    out_specs=pl.BlockSpec((BNP, 128), lambda i: (i, 0)),
    out_shape=jax.ShapeDtypeStruct((PK, 128), jnp.float32),
)


def _tc2_body(acc_ref, g_ref, dacc_ref, w2_ref, b1_ref, out_ref):
    dinv = _dinv(dacc_ref)
    s1 = jnp.maximum(dinv * (acc_ref[0] + acc_ref[1] + g_ref[...]) + b1_ref[...],
                     0.0)
    out_ref[...] = _mm(s1, w2_ref[...]) * dinv


_tc2 = pl.pallas_call(
    _tc2_body,
    grid=(PK // BNP,),
    in_specs=[
        pl.BlockSpec((NC, BNP, 128), lambda i: (0, i, 0)),
        pl.BlockSpec((BNP, 128), lambda i: (i, 0)),
        pl.BlockSpec((NC, BNP, 128), lambda i: (0, i, 0)),
        pl.BlockSpec((128, 128), lambda i: (0, 0)),
        pl.BlockSpec((1, 128), lambda i: (0, 0)),
    ],
    out_specs=pl.BlockSpec((BNP, 128), lambda i: (i, 0)),
    out_shape=jax.ShapeDtypeStruct((PK, 128), jnp.float32),
)


PKR = N // 8


def _tc3_body(acc_ref, g_ref, dacc_ref, wfc_ref, b2_ref, bfc_ref, out_ref):
    dinv = _dinv(dacc_ref)
    s2 = jnp.maximum(dinv * (acc_ref[0] + acc_ref[1] + g_ref[...]) + b2_ref[...],
                     0.0)
    out_ref[...] = (_mm(s2, wfc_ref[...]) + bfc_ref[...])[:PKR]


_tc3 = pl.pallas_call(
    _tc3_body,
    grid=(1,),
    in_specs=[
        pl.BlockSpec((NC, PK, 128), lambda i: (0, 0, 0)),
        pl.BlockSpec((PK, 128), lambda i: (0, 0)),
        pl.BlockSpec((NC, PK, 128), lambda i: (0, 0, 0)),
        pl.BlockSpec((128, 8 * C), lambda i: (0, 0)),
        pl.BlockSpec((1, 128), lambda i: (0, 0)),
        pl.BlockSpec((1, 8 * C), lambda i: (0, 0)),
    ],
    out_specs=pl.BlockSpec((PKR, 8 * C), lambda i: (0, 0)),
    out_shape=jax.ShapeDtypeStruct((PKR, 8 * C), jnp.float32),
)


def _kron8(w):
    return jnp.kron(jnp.eye(8, dtype=w.dtype), w)


@jax.jit
def kernel(x, edge_index, W1, b1, W2, b2, Wfc, bfc):
    ei = edge_index.astype(jnp.int32).reshape(2, NW, NCH, CHUNK)

    x2 = jnp.pad(x, ((0, NP - N), (0, 0))).reshape(PK, 8 * F)
    dacc = _deg_kernel(ei).reshape(NC, PK, 128)
    g1 = _tc1(x2, _kron8(W1), dacc)
    acc1 = _prop_kernel(g1.reshape(NP, H), ei).reshape(NC, PK, 128)
    g2 = _tc2(acc1, g1, dacc, _kron8(W2), jnp.tile(b1, 8).reshape(1, 128))
    acc2 = _prop_kernel(g2.reshape(NP, H), ei).reshape(NC, PK, 128)
    out = _tc3(acc2, g2, dacc, _kron8(Wfc), jnp.tile(b2, 8).reshape(1, 128),
               jnp.tile(bfc, 8).reshape(1, 8 * C))
    return out.reshape(N, C)


# speedup vs baseline: 1.0706x; 1.0706x over previous
import functools

import jax
import jax.numpy as jnp
from jax import lax
from jax.experimental import pallas as pl
from jax.experimental.pallas import tpu as pltpu
from jax.experimental.pallas import tpu_sc as plsc

N = 10000
E = 320000
F = 128
H = 16
C = 4

NC = 2
NS = 16
NW = NC * NS

CHUNK = 400
NCH = 25
EPW = NCH * CHUNK

NP = 10240
RPS = NP // NS

NBUF = 5
AHEAD = 4

_mesh = plsc.VectorSubcoreMesh(
    core_axis_name="c", subcore_axis_name="s", num_cores=NC, num_subcores=NS
)

_sc_params = pltpu.CompilerParams(use_tc_tiling_on_sc=False)


def _stripe_out(zb_v, acc_sh, out_hbm, cid, sid):
    pltpu.sync_copy(acc_sh.at[pl.ds(sid * RPS, RPS)], zb_v)
    pltpu.sync_copy(zb_v, out_hbm.at[cid, sid])


def _zero_acc(zb_v, acc_sh, sid):
    zero_row = jnp.zeros((H,), jnp.float32)

    def zfill(i, c):
        zb_v[i, :] = zero_row
        return c
    lax.fori_loop(0, RPS, zfill, 0)
    pltpu.sync_copy(zb_v, acc_sh.at[pl.ds(sid * RPS, RPS)])


@functools.partial(
    pl.kernel,
    out_type=jax.ShapeDtypeStruct((NC, NS, RPS, H), jnp.float32),
    mesh=_mesh,
    scratch_types=[
        pltpu.VMEM((NCH, CHUNK), jnp.int32),
        pltpu.VMEM((CHUNK, H), jnp.float32),
        pltpu.VMEM((RPS, H), jnp.float32),
        pltpu.VMEM_SHARED((NP, H), jnp.float32),
        pltpu.SemaphoreType.DMA,
    ],
    compiler_params=_sc_params,
)
def _deg_kernel(ei_hbm, out_hbm, coli_v, ones_v, zb_v, acc_sh, sem_s):
    cid = lax.axis_index("c")
    sid = lax.axis_index("s")
    wid = cid * NS + sid

    one_row = jnp.full((H,), 1.0, jnp.float32)

    def fill(i, c):
        ones_v[i, :] = one_row
        return c
    lax.fori_loop(0, CHUNK, fill, 0)

    _zero_acc(zb_v, acc_sh, sid)
    pltpu.sync_copy(ei_hbm.at[1, wid], coli_v)
    plsc.subcore_barrier()

    def scat(j):
        return pltpu.make_async_copy(ones_v, acc_sh.at[coli_v.at[j]], sem_s)

    LAG = 4

    def sc(j, c):
        scat(j).start(add=True)

        @pl.when(j >= LAG)
        def _():
            scat(j - LAG).wait()
        return c
    lax.fori_loop(0, NCH, sc, 0)

    def drain(j, c):
        scat(NCH - LAG + j).wait()
        return c
    lax.fori_loop(0, LAG, drain, 0)

    plsc.subcore_barrier()
    _stripe_out(zb_v, acc_sh, out_hbm, cid, sid)


@functools.partial(
    pl.kernel,
    out_type=jax.ShapeDtypeStruct((NC, NS, RPS, H), jnp.float32),
    mesh=_mesh,
    scratch_types=[
        pltpu.VMEM((NCH, CHUNK), jnp.int32),
        pltpu.VMEM((NCH, CHUNK), jnp.int32),
        [pltpu.VMEM((CHUNK, H), jnp.float32)] * NBUF,
        pltpu.VMEM((RPS, H), jnp.float32),
        pltpu.VMEM_SHARED((NP, H), jnp.float32),
        pltpu.SemaphoreType.DMA,
        pltpu.SemaphoreType.DMA,
    ],
    compiler_params=_sc_params,
)
def _prop_kernel(g_hbm, ei_hbm, out_hbm,
                 rowi_v, coli_v, bufs, zb_v, acc_sh, sem_g, sem_s):
    cid = lax.axis_index("c")
    sid = lax.axis_index("s")
    wid = cid * NS + sid

    _zero_acc(zb_v, acc_sh, sid)
    pltpu.sync_copy(ei_hbm.at[0, wid], rowi_v)
    pltpu.sync_copy(ei_hbm.at[1, wid], coli_v)
    plsc.subcore_barrier()

    def gat(j, b):
        return pltpu.make_async_copy(g_hbm.at[rowi_v.at[j]], bufs[b], sem_g)

    def scat(j, b):
        return pltpu.make_async_copy(bufs[b], acc_sh.at[coli_v.at[j]], sem_s)

    for j in range(AHEAD):
        gat(j, j).start()

    def step(g, c):
        for b in range(NBUF):
            j = g * NBUF + b
            bp = (b + AHEAD) % NBUF
            back = NBUF - AHEAD
            if b >= back:
                scat(j - back, bp).wait()
            else:
                @pl.when(j >= back)
                def _():
                    scat(j - back, bp).wait()

            @pl.when(j + AHEAD < NCH)
            def _():
                gat(j + AHEAD, bp).start()

            gat(j, b).wait()
            scat(j, b).start(add=True)
        return c
    lax.fori_loop(0, NCH // NBUF, step, 0)

    for k in range(NBUF - AHEAD):
        j = NCH - (NBUF - AHEAD) + k
        scat(j, j % NBUF).wait()

    plsc.subcore_barrier()
    _stripe_out(zb_v, acc_sh, out_hbm, cid, sid)


PK = NP // 8
BNP = PK // 2


def _mm(a, b):
    return lax.dot_general(a, b, (((1,), (0,)), ((), ())),
                           preferred_element_type=jnp.float32)


def _dinv(dacc_ref):
    return lax.rsqrt(dacc_ref[0] + dacc_ref[1] + 1.0)


def _tc1_body(x2_ref, w1_ref, dacc_ref, g_ref):
    g_ref[...] = _mm(x2_ref[...], w1_ref[...]) * _dinv(dacc_ref)


_tc1 = pl.pallas_call(
    _tc1_body,
    grid=(PK // BNP,),
    in_specs=[
        pl.BlockSpec((BNP, 8 * F), lambda i: (i, 0)),
        pl.BlockSpec((8 * F, 128), lambda i: (0, 0)),
        pl.BlockSpec((NC, BNP, 128), lambda i: (0, i, 0)),
    ],
    out_specs=pl.BlockSpec((BNP, 128), lambda i: (i, 0)),
    out_shape=jax.ShapeDtypeStruct((PK, 128), jnp.float32),
)


def _tc2_body(acc_ref, g_ref, dacc_ref, w2_ref, b1_ref, out_ref):
    dinv = _dinv(dacc_ref)
    s1 = jnp.maximum(dinv * (acc_ref[0] + acc_ref[1] + g_ref[...]) + b1_ref[...],
                     0.0)
    out_ref[...] = _mm(s1, w2_ref[...]) * dinv


_tc2 = pl.pallas_call(
    _tc2_body,
    grid=(PK // BNP,),
    in_specs=[
        pl.BlockSpec((NC, BNP, 128), lambda i: (0, i, 0)),
        pl.BlockSpec((BNP, 128), lambda i: (i, 0)),
        pl.BlockSpec((NC, BNP, 128), lambda i: (0, i, 0)),
        pl.BlockSpec((128, 128), lambda i: (0, 0)),
        pl.BlockSpec((1, 128), lambda i: (0, 0)),
    ],
    out_specs=pl.BlockSpec((BNP, 128), lambda i: (i, 0)),
    out_shape=jax.ShapeDtypeStruct((PK, 128), jnp.float32),
)


PKR = N // 8


def _tc3_body(acc_ref, g_ref, dacc_ref, wfc_ref, b2_ref, bfc_ref, out_ref):
    dinv = _dinv(dacc_ref)
    s2 = jnp.maximum(dinv * (acc_ref[0] + acc_ref[1] + g_ref[...]) + b2_ref[...],
                     0.0)
    out_ref[...] = (_mm(s2, wfc_ref[...]) + bfc_ref[...])[:PKR]


_tc3 = pl.pallas_call(
    _tc3_body,
    grid=(1,),
    in_specs=[
        pl.BlockSpec((NC, PK, 128), lambda i: (0, 0, 0)),
        pl.BlockSpec((PK, 128), lambda i: (0, 0)),
        pl.BlockSpec((NC, PK, 128), lambda i: (0, 0, 0)),
        pl.BlockSpec((128, 8 * C), lambda i: (0, 0)),
        pl.BlockSpec((1, 128), lambda i: (0, 0)),
        pl.BlockSpec((1, 8 * C), lambda i: (0, 0)),
    ],
    out_specs=pl.BlockSpec((PKR, 8 * C), lambda i: (0, 0)),
    out_shape=jax.ShapeDtypeStruct((PKR, 8 * C), jnp.float32),
)


def _kron8(w):
    return jnp.kron(jnp.eye(8, dtype=w.dtype), w)


@jax.jit
def kernel(x, edge_index, W1, b1, W2, b2, Wfc, bfc):
    ei = edge_index.astype(jnp.int32).reshape(2, NW, NCH, CHUNK)

    x2 = jnp.pad(x, ((0, NP - N), (0, 0))).reshape(PK, 8 * F)
    dacc = _deg_kernel(ei).reshape(NC, PK, 128)
    g1 = _tc1(x2, _kron8(W1), dacc)
    acc1 = _prop_kernel(g1.reshape(NP, H), ei).reshape(NC, PK, 128)
    g2 = _tc2(acc1, g1, dacc, _kron8(W2), jnp.tile(b1, 8).reshape(1, 128))
    acc2 = _prop_kernel(g2.reshape(NP, H), ei).reshape(NC, PK, 128)
    out = _tc3(acc2, g2, dacc, _kron8(Wfc), jnp.tile(b2, 8).reshape(1, 128),
               jnp.tile(bfc, 8).reshape(1, 8 * C))
    return out.reshape(N, C)


# speedup vs baseline: 1.1723x; 1.0950x over previous
import functools

import jax
import jax.numpy as jnp
from jax import lax
from jax.experimental import pallas as pl
from jax.experimental.pallas import tpu as pltpu
from jax.experimental.pallas import tpu_sc as plsc

N = 10000
E = 320000
F = 128
H = 16
C = 4

NC = 2
NS = 16
NW = NC * NS

CHUNK = 400
NCH = 25
EPW = NCH * CHUNK

NP = 10240
RPS = NP // NS

NBUF = 5
AHEAD = 4

_mesh = plsc.VectorSubcoreMesh(
    core_axis_name="c", subcore_axis_name="s", num_cores=NC, num_subcores=NS
)

_sc_params = pltpu.CompilerParams(use_tc_tiling_on_sc=False)


def _stripe_out(zb_v, acc_sh, out_hbm, cid, sid):
    pltpu.sync_copy(acc_sh.at[pl.ds(sid * RPS, RPS)], zb_v)
    pltpu.sync_copy(zb_v, out_hbm.at[cid, sid])


LAGI = 8


def _idx_desc(ei_hbm, dst_v, axis, base, j, sem):
    return pltpu.make_async_copy(
        ei_hbm.at[axis, pl.ds(base + j * CHUNK, CHUNK)], dst_v.at[j], sem)


def _zero_acc(zb_v, acc_sh, sid):
    zero_row = jnp.zeros((H,), jnp.float32)

    def zfill(i, c):
        zb_v[i, :] = zero_row
        return c
    lax.fori_loop(0, RPS, zfill, 0)
    pltpu.sync_copy(zb_v, acc_sh.at[pl.ds(sid * RPS, RPS)])


@functools.partial(
    pl.kernel,
    out_type=jax.ShapeDtypeStruct((NC, NS, RPS, H), jnp.float32),
    mesh=_mesh,
    scratch_types=[
        pltpu.VMEM((NCH, CHUNK), jnp.int32),
        pltpu.VMEM((CHUNK, H), jnp.float32),
        pltpu.VMEM((RPS, H), jnp.float32),
        pltpu.VMEM_SHARED((NP, H), jnp.float32),
        pltpu.SemaphoreType.DMA,
    ],
    compiler_params=_sc_params,
)
def _deg_kernel(ei_hbm, out_hbm, coli_v, ones_v, zb_v, acc_sh, sem_s):
    cid = lax.axis_index("c")
    sid = lax.axis_index("s")
    wid = cid * NS + sid

    one_row = jnp.full((H,), 1.0, jnp.float32)

    def fill(i, c):
        ones_v[i, :] = one_row
        return c
    lax.fori_loop(0, CHUNK, fill, 0)

    base = wid * EPW

    def ldf(j, c):
        _idx_desc(ei_hbm, coli_v, 1, base, j, sem_s).start()
        return c
    lax.fori_loop(0, NCH, ldf, 0)

    _zero_acc(zb_v, acc_sh, sid)

    def ldw(j, c):
        _idx_desc(ei_hbm, coli_v, 1, base, j, sem_s).wait()
        return c
    lax.fori_loop(0, NCH, ldw, 0)
    plsc.subcore_barrier()

    def scat(j):
        return pltpu.make_async_copy(ones_v, acc_sh.at[coli_v.at[j]], sem_s)

    LAG = 8

    def sc(j, c):
        scat(j).start(add=True)

        @pl.when(j >= LAG)
        def _():
            scat(j - LAG).wait()
        return c
    lax.fori_loop(0, NCH, sc, 0)

    def drain(j, c):
        scat(NCH - LAG + j).wait()
        return c
    lax.fori_loop(0, LAG, drain, 0)

    plsc.subcore_barrier()
    _stripe_out(zb_v, acc_sh, out_hbm, cid, sid)


@functools.partial(
    pl.kernel,
    out_type=jax.ShapeDtypeStruct((NC, NS, RPS, H), jnp.float32),
    mesh=_mesh,
    scratch_types=[
        pltpu.VMEM((NCH, CHUNK), jnp.int32),
        pltpu.VMEM((NCH, CHUNK), jnp.int32),
        [pltpu.VMEM((CHUNK, H), jnp.float32)] * NBUF,
        pltpu.VMEM((RPS, H), jnp.float32),
        pltpu.VMEM_SHARED((NP, H), jnp.float32),
        pltpu.SemaphoreType.DMA,
        pltpu.SemaphoreType.DMA,
    ],
    compiler_params=_sc_params,
)
def _prop_kernel(g_hbm, ei_hbm, out_hbm,
                 rowi_v, coli_v, bufs, zb_v, acc_sh, sem_g, sem_s):
    cid = lax.axis_index("c")
    sid = lax.axis_index("s")
    wid = cid * NS + sid

    base = wid * EPW

    def ldf(j, c):
        _idx_desc(ei_hbm, rowi_v, 0, base, j, sem_g).start()
        _idx_desc(ei_hbm, coli_v, 1, base, j, sem_g).start()
        return c
    lax.fori_loop(0, NCH, ldf, 0)

    _zero_acc(zb_v, acc_sh, sid)

    def ldw(j, c):
        _idx_desc(ei_hbm, rowi_v, 0, base, j, sem_g).wait()
        _idx_desc(ei_hbm, coli_v, 1, base, j, sem_g).wait()
        return c
    lax.fori_loop(0, NCH, ldw, 0)
    plsc.subcore_barrier()

    def gat(j, b):
        return pltpu.make_async_copy(g_hbm.at[rowi_v.at[j]], bufs[b], sem_g)

    def scat(j, b):
        return pltpu.make_async_copy(bufs[b], acc_sh.at[coli_v.at[j]], sem_s)

    for j in range(AHEAD):
        gat(j, j).start()

    def step(g, c):
        for b in range(NBUF):
            j = g * NBUF + b
            bp = (b + AHEAD) % NBUF
            back = NBUF - AHEAD
            if b >= back:
                scat(j - back, bp).wait()
            else:
                @pl.when(j >= back)
                def _():
                    scat(j - back, bp).wait()

            @pl.when(j + AHEAD < NCH)
            def _():
                gat(j + AHEAD, bp).start()

            gat(j, b).wait()
            scat(j, b).start(add=True)
        return c
    lax.fori_loop(0, NCH // NBUF, step, 0)

    for k in range(NBUF - AHEAD):
        j = NCH - (NBUF - AHEAD) + k
        scat(j, j % NBUF).wait()

    plsc.subcore_barrier()
    _stripe_out(zb_v, acc_sh, out_hbm, cid, sid)


PK = NP // 8
BNP = PK // 2


def _mm(a, b):
    return lax.dot_general(a, b, (((1,), (0,)), ((), ())),
                           preferred_element_type=jnp.float32)


def _dinv(dacc_ref):
    return lax.rsqrt(dacc_ref[0] + dacc_ref[1] + 1.0)


def _tc1a_body(x2_ref, w1_ref, h_ref):
    h_ref[...] = _mm(x2_ref[...], w1_ref[...])


_tc1a = pl.pallas_call(
    _tc1a_body,
    grid=(PK // BNP,),
    in_specs=[
        pl.BlockSpec((BNP, 8 * F), lambda i: (i, 0)),
        pl.BlockSpec((8 * F, 128), lambda i: (0, 0)),
    ],
    out_specs=pl.BlockSpec((BNP, 128), lambda i: (i, 0)),
    out_shape=jax.ShapeDtypeStruct((PK, 128), jnp.float32),
)


def _tc1b_body(h_ref, dacc_ref, g_ref):
    g_ref[...] = h_ref[...] * _dinv(dacc_ref)


_tc1b = pl.pallas_call(
    _tc1b_body,
    grid=(PK // BNP,),
    in_specs=[
        pl.BlockSpec((BNP, 128), lambda i: (i, 0)),
        pl.BlockSpec((NC, BNP, 128), lambda i: (0, i, 0)),
    ],
    out_specs=pl.BlockSpec((BNP, 128), lambda i: (i, 0)),
    out_shape=jax.ShapeDtypeStruct((PK, 128), jnp.float32),
)


def _tc2_body(acc_ref, g_ref, dacc_ref, w2_ref, b1_ref, out_ref):
    dinv = _dinv(dacc_ref)
    s1 = jnp.maximum(dinv * (acc_ref[0] + acc_ref[1] + g_ref[...]) + b1_ref[...],
                     0.0)
    out_ref[...] = _mm(s1, w2_ref[...]) * dinv


_tc2 = pl.pallas_call(
    _tc2_body,
    grid=(PK // BNP,),
    in_specs=[
        pl.BlockSpec((NC, BNP, 128), lambda i: (0, i, 0)),
        pl.BlockSpec((BNP, 128), lambda i: (i, 0)),
        pl.BlockSpec((NC, BNP, 128), lambda i: (0, i, 0)),
        pl.BlockSpec((128, 128), lambda i: (0, 0)),
        pl.BlockSpec((1, 128), lambda i: (0, 0)),
    ],
    out_specs=pl.BlockSpec((BNP, 128), lambda i: (i, 0)),
    out_shape=jax.ShapeDtypeStruct((PK, 128), jnp.float32),
)


PKR = N // 8


def _tc3_body(acc_ref, g_ref, dacc_ref, wfc_ref, b2_ref, bfc_ref, out_ref):
    dinv = _dinv(dacc_ref)
    s2 = jnp.maximum(dinv * (acc_ref[0] + acc_ref[1] + g_ref[...]) + b2_ref[...],
                     0.0)
    out_ref[...] = (_mm(s2, wfc_ref[...]) + bfc_ref[...])[:PKR]


_tc3 = pl.pallas_call(
    _tc3_body,
    grid=(1,),
    in_specs=[
        pl.BlockSpec((NC, PK, 128), lambda i: (0, 0, 0)),
        pl.BlockSpec((PK, 128), lambda i: (0, 0)),
        pl.BlockSpec((NC, PK, 128), lambda i: (0, 0, 0)),
        pl.BlockSpec((128, 8 * C), lambda i: (0, 0)),
        pl.BlockSpec((1, 128), lambda i: (0, 0)),
        pl.BlockSpec((1, 8 * C), lambda i: (0, 0)),
    ],
    out_specs=pl.BlockSpec((PKR, 8 * C), lambda i: (0, 0)),
    out_shape=jax.ShapeDtypeStruct((PKR, 8 * C), jnp.float32),
)


def _kron8(w):
    return jnp.kron(jnp.eye(8, dtype=w.dtype), w)


@jax.jit
def kernel(x, edge_index, W1, b1, W2, b2, Wfc, bfc):
    ei = edge_index.astype(jnp.int32)

    x2 = jnp.pad(x, ((0, NP - N), (0, 0))).reshape(PK, 8 * F)
    dacc = _deg_kernel(ei).reshape(NC, PK, 128)
    g1 = _tc1b(_tc1a(x2, _kron8(W1)), dacc)
    acc1 = _prop_kernel(g1.reshape(NP, H), ei).reshape(NC, PK, 128)
    g2 = _tc2(acc1, g1, dacc, _kron8(W2), jnp.tile(b1, 8).reshape(1, 128))
    acc2 = _prop_kernel(g2.reshape(NP, H), ei).reshape(NC, PK, 128)
    out = _tc3(acc2, g2, dacc, _kron8(Wfc), jnp.tile(b2, 8).reshape(1, 128),
               jnp.tile(bfc, 8).reshape(1, 8 * C))
    return out.reshape(N, C)
